# baseline (speedup 1.0000x reference)
import functools

import jax
import jax.numpy as jnp
from jax.experimental import pallas as pl
from jax.experimental.pallas import tpu as pltpu

B = 16
M = 2048
D = 256
H = 8
A = 64
O = 64
HA = H * A
HO = H * O
TM = 512
NT = M // TM
NEG = -1e30


def _body(xf_ref, mask_ref, Ws_ref, bs_ref, Wq_ref, bq_ref, Wk_ref, bk_ref,
          Wv_ref, bv_ref, out_ref, frac_ref,
          zmax_ref, den_ref, acc_ref, en_ref):
    t = pl.program_id(1)

    @pl.when(t == 0)
    def _init():
        zmax_ref[...] = jnp.zeros_like(zmax_ref)
        den_ref[...] = jnp.zeros_like(den_ref)
        acc_ref[...] = jnp.zeros_like(acc_ref)
        en_ref[0, 0] = 0.0

    xf = xf_ref[0].astype(jnp.bfloat16).reshape(TM, D)
    activ = jnp.dot(xf, Ws_ref[...],
                    preferred_element_type=jnp.float32) + bs_ref[...]
    activ_b = activ.astype(jnp.bfloat16)
    q = jnp.dot(activ_b, Wq_ref[...],
                preferred_element_type=jnp.float32) + bq_ref[...]
    k = jnp.dot(activ_b, Wk_ref[...],
                preferred_element_type=jnp.float32) + bk_ref[...]
    v = jnp.dot(activ_b, Wv_ref[...],
                preferred_element_type=jnp.float32) + bv_ref[...]

    qk = q * k
    lane = jax.lax.broadcasted_iota(jnp.int32, (HA, H), 0)
    head = jax.lax.broadcasted_iota(jnp.int32, (HA, H), 1)
    sel = (lane // A == head).astype(jnp.float32)
    scores = jnp.dot(qk, sel,
                     preferred_element_type=jnp.float32) * (1.0 / (A ** 0.5))

    m = mask_ref[0]
    en_ref[0, 0] += jnp.sum(m)
    z = jnp.where(m > 0.5, scores, NEG)

    old_max = zmax_ref[...]
    new_max = jnp.maximum(old_max, jnp.max(z, axis=0, keepdims=True))
    zmax_ref[...] = new_max
    scale = jnp.exp(old_max - new_max)
    ez = jnp.exp(z - new_max)
    den_ref[...] = den_ref[...] * scale + jnp.sum(ez, axis=0, keepdims=True)
    contrib = jax.lax.dot_general(ez, v, (((0,), (0,)), ((), ())),
                                  preferred_element_type=jnp.float32)
    acc_ref[...] = acc_ref[...] * scale.T + contrib

    @pl.when(t == NT - 1)
    def _fin():
        den = den_ref[...] + 1.0
        norm = acc_ref[...] / den.T
        row = jax.lax.broadcasted_iota(jnp.int32, (H, HO), 0)
        col = jax.lax.broadcasted_iota(jnp.int32, (H, HO), 1)
        pick = (col // O == row).astype(jnp.float32)
        out_ref[0] = jnp.sum(norm * pick, axis=0, keepdims=True)
        frac_ref[0] = jnp.broadcast_to(en_ref[0, 0] * (1.0 / M), (1, 1))


@jax.jit
def kernel(x, Ws, bs, Wq, bq, Wk, bk, Wv, bv):
    xr = x.reshape(B, M * D // 2048 + 1, 2048)
    mask = x[:, M * D:].reshape(B, M, 1)
    grid = (B, NT)
    out_main, frac = pl.pallas_call(
        _body,
        grid=grid,
        in_specs=[
            pl.BlockSpec((1, TM // 8, 2048), lambda b, t: (b, t, 0)),
            pl.BlockSpec((1, TM, 1), lambda b, t: (b, t, 0)),
            pl.BlockSpec((D, D), lambda b, t: (0, 0)),
            pl.BlockSpec((1, D), lambda b, t: (0, 0)),
            pl.BlockSpec((D, HA), lambda b, t: (0, 0)),
            pl.BlockSpec((1, HA), lambda b, t: (0, 0)),
            pl.BlockSpec((D, HA), lambda b, t: (0, 0)),
            pl.BlockSpec((1, HA), lambda b, t: (0, 0)),
            pl.BlockSpec((D, HO), lambda b, t: (0, 0)),
            pl.BlockSpec((1, HO), lambda b, t: (0, 0)),
        ],
        out_specs=[
            pl.BlockSpec((1, 1, HO), lambda b, t: (b, 0, 0)),
            pl.BlockSpec((1, 1, 1), lambda b, t: (b, 0, 0)),
        ],
        out_shape=[
            jax.ShapeDtypeStruct((B, 1, HO), jnp.float32),
            jax.ShapeDtypeStruct((B, 1, 1), jnp.float32),
        ],
        scratch_shapes=[
            pltpu.VMEM((1, H), jnp.float32),
            pltpu.VMEM((1, H), jnp.float32),
            pltpu.VMEM((H, HO), jnp.float32),
            pltpu.SMEM((1, 1), jnp.float32),
        ],
    )(xr, mask, Ws.astype(jnp.bfloat16), bs.reshape(1, D),
      Wq.astype(jnp.bfloat16), bq.reshape(1, HA),
      Wk.astype(jnp.bfloat16), bk.reshape(1, HA),
      Wv.astype(jnp.bfloat16), bv.reshape(1, HO))
    return jnp.concatenate([out_main.reshape(B, HO), frac.reshape(B, 1)],
                           axis=1)


# speedup vs baseline: 1.0156x; 1.0156x over previous
import jax
import jax.numpy as jnp
from jax.experimental import pallas as pl
from jax.experimental.pallas import tpu as pltpu

B = 16
M = 2048
D = 256
H = 8
A = 64
O = 64
HA = H * A
HO = H * O
TM = 512
NT = M // TM
NEG = -1e30


def _body(xf_ref, mask_ref, Ws_ref, bs_ref, Wq_ref, bq_ref, Wk_ref, bk_ref,
          Wv_ref, bv_ref, out_ref, frac_ref,
          s_buf, v_buf, en_ref):
    t = pl.program_id(1)

    @pl.when(t == 0)
    def _init():
        en_ref[0, 0] = 0.0

    xf = xf_ref[0].astype(jnp.bfloat16).reshape(TM, D)
    activ = jnp.dot(xf, Ws_ref[...],
                    preferred_element_type=jnp.float32) + bs_ref[...]
    activ_b = activ.astype(jnp.bfloat16)
    q = jnp.dot(activ_b, Wq_ref[...],
                preferred_element_type=jnp.float32) + bq_ref[...]
    k = jnp.dot(activ_b, Wk_ref[...],
                preferred_element_type=jnp.float32) + bk_ref[...]
    v = jnp.dot(activ_b, Wv_ref[...],
                preferred_element_type=jnp.float32) + bv_ref[...]

    qk = q * k
    lane = jax.lax.broadcasted_iota(jnp.int32, (HA, H), 0)
    head = jax.lax.broadcasted_iota(jnp.int32, (HA, H), 1)
    sel = (lane // A == head).astype(jnp.float32)
    scores = jnp.dot(qk, sel,
                     preferred_element_type=jnp.float32) * (1.0 / (A ** 0.5))

    m = mask_ref[0]
    en_ref[0, 0] += jnp.sum(m)
    s_buf[pl.ds(t * TM, TM), :] = jnp.where(m > 0.5, scores, NEG)
    v_buf[pl.ds(t * TM, TM), :] = v.astype(jnp.bfloat16)

    @pl.when(t == NT - 1)
    def _fin():
        s = s_buf[...]
        zmax = jnp.maximum(jnp.max(s, axis=0, keepdims=True), 0.0)
        ez = jnp.exp(s - zmax)
        den = jnp.sum(ez, axis=0, keepdims=True) + 1.0
        attn = (ez / den).astype(jnp.bfloat16)
        acc = jax.lax.dot_general(attn, v_buf[...], (((0,), (0,)), ((), ())),
                                  preferred_element_type=jnp.float32)
        row = jax.lax.broadcasted_iota(jnp.int32, (H, HO), 0)
        col = jax.lax.broadcasted_iota(jnp.int32, (H, HO), 1)
        pick = (col // O == row).astype(jnp.float32)
        out_ref[0] = jnp.sum(acc * pick, axis=0, keepdims=True)
        frac_ref[0] = jnp.broadcast_to(en_ref[0, 0] * (1.0 / M), (1, 1))


@jax.jit
def kernel(x, Ws, bs, Wq, bq, Wk, bk, Wv, bv):
    xr = x.reshape(B, M * D // 2048 + 1, 2048)
    mask = x[:, M * D:].reshape(B, M, 1)
    grid = (B, NT)
    out_main, frac = pl.pallas_call(
        _body,
        grid=grid,
        in_specs=[
            pl.BlockSpec((1, TM // 8, 2048), lambda b, t: (b, t, 0)),
            pl.BlockSpec((1, TM, 1), lambda b, t: (b, t, 0)),
            pl.BlockSpec((D, D), lambda b, t: (0, 0)),
            pl.BlockSpec((1, D), lambda b, t: (0, 0)),
            pl.BlockSpec((D, HA), lambda b, t: (0, 0)),
            pl.BlockSpec((1, HA), lambda b, t: (0, 0)),
            pl.BlockSpec((D, HA), lambda b, t: (0, 0)),
            pl.BlockSpec((1, HA), lambda b, t: (0, 0)),
            pl.BlockSpec((D, HO), lambda b, t: (0, 0)),
            pl.BlockSpec((1, HO), lambda b, t: (0, 0)),
        ],
        out_specs=[
            pl.BlockSpec((1, 1, HO), lambda b, t: (b, 0, 0)),
            pl.BlockSpec((1, 1, 1), lambda b, t: (b, 0, 0)),
        ],
        out_shape=[
            jax.ShapeDtypeStruct((B, 1, HO), jnp.float32),
            jax.ShapeDtypeStruct((B, 1, 1), jnp.float32),
        ],
        scratch_shapes=[
            pltpu.VMEM((M, H), jnp.float32),
            pltpu.VMEM((M, HO), jnp.bfloat16),
            pltpu.SMEM((1, 1), jnp.float32),
        ],
    )(xr, mask, Ws.astype(jnp.bfloat16), bs.reshape(1, D),
      Wq.astype(jnp.bfloat16), bq.reshape(1, HA),
      Wk.astype(jnp.bfloat16), bk.reshape(1, HA),
      Wv.astype(jnp.bfloat16), bv.reshape(1, HO))
    return jnp.concatenate([out_main.reshape(B, HO), frac.reshape(B, 1)],
                           axis=1)


# speedup vs baseline: 1.0354x; 1.0196x over previous
import jax
import jax.numpy as jnp
from jax.experimental import pallas as pl
from jax.experimental.pallas import tpu as pltpu

B = 16
M = 2048
D = 256
H = 8
A = 64
O = 64
HA = H * A
HO = H * O
TM = 512
NT = M // TM
NEG = -1e30


def _body(xf_ref, mask_ref, Ws_ref, bs_ref, Wq_ref, bq_ref, Wk_ref, bk_ref,
          Wv_ref, bv_ref, out_ref, frac_ref,
          s_buf, a_buf, en_ref):
    t = pl.program_id(1)

    @pl.when(t == 0)
    def _init():
        en_ref[0, 0] = 0.0

    xf = xf_ref[0].astype(jnp.bfloat16).reshape(TM, D)
    activ = jnp.dot(xf, Ws_ref[...],
                    preferred_element_type=jnp.float32) + bs_ref[...]
    activ_b = activ.astype(jnp.bfloat16)
    a_buf[pl.ds(t * TM, TM), :] = activ_b
    q = jnp.dot(activ_b, Wq_ref[...],
                preferred_element_type=jnp.float32) + bq_ref[...]
    k = jnp.dot(activ_b, Wk_ref[...],
                preferred_element_type=jnp.float32) + bk_ref[...]

    qk = q * k
    lane = jax.lax.broadcasted_iota(jnp.int32, (HA, H), 0)
    head = jax.lax.broadcasted_iota(jnp.int32, (HA, H), 1)
    sel = (lane // A == head).astype(jnp.float32)
    scores = jnp.dot(qk, sel,
                     preferred_element_type=jnp.float32) * (1.0 / (A ** 0.5))

    m = mask_ref[0]
    en_ref[0, 0] += jnp.sum(m)
    s_buf[pl.ds(t * TM, TM), :] = jnp.where(m > 0.5, scores, NEG)

    @pl.when(t == NT - 1)
    def _fin():
        s = s_buf[...]
        zmax = jnp.maximum(jnp.max(s, axis=0, keepdims=True), 0.0)
        ez = jnp.exp(s - zmax)
        den = jnp.sum(ez, axis=0, keepdims=True) + 1.0
        attn = (ez / den).astype(jnp.bfloat16)
        ta = jax.lax.dot_general(attn, a_buf[...], (((0,), (0,)), ((), ())),
                                 preferred_element_type=jnp.float32)
        full = jnp.dot(ta.astype(jnp.bfloat16), Wv_ref[...],
                       preferred_element_type=jnp.float32)
        sa = ((den - 1.0) / den).reshape(H, 1)
        full = full + sa * bv_ref[...]
        row = jax.lax.broadcasted_iota(jnp.int32, (H, HO), 0)
        col = jax.lax.broadcasted_iota(jnp.int32, (H, HO), 1)
        pick = (col // O == row).astype(jnp.float32)
        out_ref[0] = jnp.sum(full * pick, axis=0, keepdims=True)
        frac_ref[0] = jnp.broadcast_to(en_ref[0, 0] * (1.0 / M), (1, 1))


@jax.jit
def kernel(x, Ws, bs, Wq, bq, Wk, bk, Wv, bv):
    xr = x.reshape(B, M * D // 2048 + 1, 2048)
    mask = x[:, M * D:].reshape(B, M, 1)
    grid = (B, NT)
    out_main, frac = pl.pallas_call(
        _body,
        grid=grid,
        in_specs=[
            pl.BlockSpec((1, TM // 8, 2048), lambda b, t: (b, t, 0)),
            pl.BlockSpec((1, TM, 1), lambda b, t: (b, t, 0)),
            pl.BlockSpec((D, D), lambda b, t: (0, 0)),
            pl.BlockSpec((1, D), lambda b, t: (0, 0)),
            pl.BlockSpec((D, HA), lambda b, t: (0, 0)),
            pl.BlockSpec((1, HA), lambda b, t: (0, 0)),
            pl.BlockSpec((D, HA), lambda b, t: (0, 0)),
            pl.BlockSpec((1, HA), lambda b, t: (0, 0)),
            pl.BlockSpec((D, HO), lambda b, t: (0, 0)),
            pl.BlockSpec((1, HO), lambda b, t: (0, 0)),
        ],
        out_specs=[
            pl.BlockSpec((1, 1, HO), lambda b, t: (b, 0, 0)),
            pl.BlockSpec((1, 1, 1), lambda b, t: (b, 0, 0)),
        ],
        out_shape=[
            jax.ShapeDtypeStruct((B, 1, HO), jnp.float32),
            jax.ShapeDtypeStruct((B, 1, 1), jnp.float32),
        ],
        scratch_shapes=[
            pltpu.VMEM((M, H), jnp.float32),
            pltpu.VMEM((M, D), jnp.bfloat16),
            pltpu.SMEM((1, 1), jnp.float32),
        ],
    )(xr, mask, Ws.astype(jnp.bfloat16), bs.reshape(1, D),
      Wq.astype(jnp.bfloat16), bq.reshape(1, HA),
      Wk.astype(jnp.bfloat16), bk.reshape(1, HA),
      Wv.astype(jnp.bfloat16), bv.reshape(1, HO))
    return jnp.concatenate([out_main.reshape(B, HO), frac.reshape(B, 1)],
                           axis=1)


# speedup vs baseline: 1.0943x; 1.0568x over previous
import jax
import jax.numpy as jnp
from jax.experimental import pallas as pl
from jax.experimental.pallas import tpu as pltpu

B = 16
M = 2048
D = 256
H = 8
A = 64
O = 64
HA = H * A
HO = H * O
TM = 512
NT = M // TM
NEG = -1e30


def _body(xf_ref, mask_ref, Ws_ref, bs_ref, Wq_ref, bq_ref, Wk_ref, bk_ref,
          Wv_ref, bv_ref, out_ref, frac_ref,
          s_buf, a_buf, en_ref):
    t = pl.program_id(1)

    @pl.when(t == 0)
    def _init():
        en_ref[0, 0] = 0.0

    xf = xf_ref[0]
    activ = jnp.dot(xf, Ws_ref[...],
                    preferred_element_type=jnp.float32) + bs_ref[...]
    activ_b = activ.astype(jnp.bfloat16)
    a_buf[pl.ds(t * TM, TM), :] = activ_b
    q = jnp.dot(activ_b, Wq_ref[...],
                preferred_element_type=jnp.float32) + bq_ref[...]
    k = jnp.dot(activ_b, Wk_ref[...],
                preferred_element_type=jnp.float32) + bk_ref[...]

    qk = (q * k).astype(jnp.bfloat16)
    lane = jax.lax.broadcasted_iota(jnp.int32, (HA, H), 0)
    head = jax.lax.broadcasted_iota(jnp.int32, (HA, H), 1)
    sel = (lane // A == head).astype(jnp.bfloat16)
    scores = jnp.dot(qk, sel,
                     preferred_element_type=jnp.float32) * (1.0 / (A ** 0.5))

    m = mask_ref[0]
    en_ref[0, 0] += jnp.sum(m)
    s_buf[pl.ds(t * TM, TM), :] = jnp.where(m > 0.5, scores, NEG)

    @pl.when(t == NT - 1)
    def _fin():
        s = s_buf[...]
        zmax = jnp.maximum(jnp.max(s, axis=0, keepdims=True), 0.0)
        ez = jnp.exp(s - zmax)
        den = jnp.sum(ez, axis=0, keepdims=True) + 1.0
        attn = (ez / den).astype(jnp.bfloat16)
        ta = jax.lax.dot_general(attn, a_buf[...], (((0,), (0,)), ((), ())),
                                 preferred_element_type=jnp.float32)
        full = jnp.dot(ta.astype(jnp.bfloat16), Wv_ref[...],
                       preferred_element_type=jnp.float32)
        sa = ((den - 1.0) / den).reshape(H, 1)
        full = full + sa * bv_ref[...]
        row = jax.lax.broadcasted_iota(jnp.int32, (H, HO), 0)
        col = jax.lax.broadcasted_iota(jnp.int32, (H, HO), 1)
        pick = (col // O == row).astype(jnp.float32)
        out_ref[0] = jnp.sum(full * pick, axis=0, keepdims=True)
        frac_ref[0] = jnp.broadcast_to(en_ref[0, 0] * (1.0 / M), (1, 1))


@jax.jit
def kernel(x, Ws, bs, Wq, bq, Wk, bk, Wv, bv):
    xf = x[:, : M * D].reshape(B, M, D).astype(jnp.bfloat16)
    mask = x[:, M * D:].reshape(B, M, 1)
    grid = (B, NT)
    out_main, frac = pl.pallas_call(
        _body,
        grid=grid,
        in_specs=[
            pl.BlockSpec((1, TM, D), lambda b, t: (b, t, 0)),
            pl.BlockSpec((1, TM, 1), lambda b, t: (b, t, 0)),
            pl.BlockSpec((D, D), lambda b, t: (0, 0)),
            pl.BlockSpec((1, D), lambda b, t: (0, 0)),
            pl.BlockSpec((D, HA), lambda b, t: (0, 0)),
            pl.BlockSpec((1, HA), lambda b, t: (0, 0)),
            pl.BlockSpec((D, HA), lambda b, t: (0, 0)),
            pl.BlockSpec((1, HA), lambda b, t: (0, 0)),
            pl.BlockSpec((D, HO), lambda b, t: (0, 0)),
            pl.BlockSpec((1, HO), lambda b, t: (0, 0)),
        ],
        out_specs=[
            pl.BlockSpec((1, 1, HO), lambda b, t: (b, 0, 0)),
            pl.BlockSpec((1, 1, 1), lambda b, t: (b, 0, 0)),
        ],
        out_shape=[
            jax.ShapeDtypeStruct((B, 1, HO), jnp.float32),
            jax.ShapeDtypeStruct((B, 1, 1), jnp.float32),
        ],
        scratch_shapes=[
            pltpu.VMEM((M, H), jnp.float32),
            pltpu.VMEM((M, D), jnp.bfloat16),
            pltpu.SMEM((1, 1), jnp.float32),
        ],
    )(xf, mask, Ws.astype(jnp.bfloat16), bs.reshape(1, D),
      Wq.astype(jnp.bfloat16), bq.reshape(1, HA),
      Wk.astype(jnp.bfloat16), bk.reshape(1, HA),
      Wv.astype(jnp.bfloat16), bv.reshape(1, HO))
    return jnp.concatenate([out_main.reshape(B, HO), frac.reshape(B, 1)],
                           axis=1)


# speedup vs baseline: 1.2448x; 1.1376x over previous
import jax
import jax.numpy as jnp
from jax.experimental import pallas as pl
from jax.experimental.pallas import tpu as pltpu

B = 16
M = 2048
D = 256
H = 8
A = 64
O = 64
HA = H * A
HO = H * O
TM = 1024
NT = M // TM
NEG = -1e30


def _body(xf_ref, mask_ref, Ws_ref, bs_ref, Wq_ref, bq_ref, Wk_ref, bk_ref,
          Wv_ref, bv_ref, out_ref, frac_ref,
          s_buf, a_buf, en_ref):
    t = pl.program_id(1)

    @pl.when(t == 0)
    def _init():
        en_ref[0, 0] = 0.0

    xf = xf_ref[0]
    activ = jnp.dot(xf, Ws_ref[...],
                    preferred_element_type=jnp.float32) + bs_ref[...]
    activ_b = activ.astype(jnp.bfloat16)
    a_buf[pl.ds(t * TM, TM), :] = activ_b
    q = jnp.dot(activ_b, Wq_ref[...],
                preferred_element_type=jnp.float32) + bq_ref[...]
    k = jnp.dot(activ_b, Wk_ref[...],
                preferred_element_type=jnp.float32) + bk_ref[...]

    qk = (q * k).astype(jnp.bfloat16)
    lane = jax.lax.broadcasted_iota(jnp.int32, (HA, H), 0)
    head = jax.lax.broadcasted_iota(jnp.int32, (HA, H), 1)
    sel = (lane // A == head).astype(jnp.bfloat16)
    scores = jnp.dot(qk, sel,
                     preferred_element_type=jnp.float32) * (1.0 / (A ** 0.5))

    m = mask_ref[0]
    en_ref[0, 0] += jnp.sum(m)
    s_buf[pl.ds(t * TM, TM), :] = jnp.where(m > 0.5, scores, NEG)

    @pl.when(t == NT - 1)
    def _fin():
        s = s_buf[...]
        zmax = jnp.maximum(jnp.max(s, axis=0, keepdims=True), 0.0)
        ez = jnp.exp(s - zmax)
        den = jnp.sum(ez, axis=0, keepdims=True) + 1.0
        attn = (ez / den).astype(jnp.bfloat16)
        ta = jax.lax.dot_general(attn, a_buf[...], (((0,), (0,)), ((), ())),
                                 preferred_element_type=jnp.float32)
        full = jnp.dot(ta.astype(jnp.bfloat16), Wv_ref[...],
                       preferred_element_type=jnp.float32)
        sa = ((den - 1.0) / den).reshape(H, 1)
        full = full + sa * bv_ref[...]
        row = jax.lax.broadcasted_iota(jnp.int32, (H, HO), 0)
        col = jax.lax.broadcasted_iota(jnp.int32, (H, HO), 1)
        pick = (col // O == row).astype(jnp.float32)
        out_ref[0] = jnp.sum(full * pick, axis=0, keepdims=True)
        frac_ref[0] = jnp.broadcast_to(en_ref[0, 0] * (1.0 / M), (1, 1))


@jax.jit
def kernel(x, Ws, bs, Wq, bq, Wk, bk, Wv, bv):
    xf = x[:, : M * D].reshape(B, M, D).astype(jnp.bfloat16)
    mask = x[:, M * D:].reshape(B, M, 1)
    grid = (B, NT)
    out_main, frac = pl.pallas_call(
        _body,
        grid=grid,
        in_specs=[
            pl.BlockSpec((1, TM, D), lambda b, t: (b, t, 0)),
            pl.BlockSpec((1, TM, 1), lambda b, t: (b, t, 0)),
            pl.BlockSpec((D, D), lambda b, t: (0, 0)),
            pl.BlockSpec((1, D), lambda b, t: (0, 0)),
            pl.BlockSpec((D, HA), lambda b, t: (0, 0)),
            pl.BlockSpec((1, HA), lambda b, t: (0, 0)),
            pl.BlockSpec((D, HA), lambda b, t: (0, 0)),
            pl.BlockSpec((1, HA), lambda b, t: (0, 0)),
            pl.BlockSpec((D, HO), lambda b, t: (0, 0)),
            pl.BlockSpec((1, HO), lambda b, t: (0, 0)),
        ],
        out_specs=[
            pl.BlockSpec((1, 1, HO), lambda b, t: (b, 0, 0)),
            pl.BlockSpec((1, 1, 1), lambda b, t: (b, 0, 0)),
        ],
        out_shape=[
            jax.ShapeDtypeStruct((B, 1, HO), jnp.float32),
            jax.ShapeDtypeStruct((B, 1, 1), jnp.float32),
        ],
        scratch_shapes=[
            pltpu.VMEM((M, H), jnp.float32),
            pltpu.VMEM((M, D), jnp.bfloat16),
            pltpu.SMEM((1, 1), jnp.float32),
        ],
    )(xf, mask, Ws.astype(jnp.bfloat16), bs.reshape(1, D),
      Wq.astype(jnp.bfloat16), bq.reshape(1, HA),
      Wk.astype(jnp.bfloat16), bk.reshape(1, HA),
      Wv.astype(jnp.bfloat16), bv.reshape(1, HO))
    return jnp.concatenate([out_main.reshape(B, HO), frac.reshape(B, 1)],
                           axis=1)


# speedup vs baseline: 1.2910x; 1.0371x over previous
import jax
import jax.numpy as jnp
from jax.experimental import pallas as pl
from jax.experimental.pallas import tpu as pltpu

B = 16
M = 2048
D = 256
H = 8
A = 64
O = 64
HA = H * A
HO = H * O
TM = 2048
NT = M // TM
NEG = -1e30


def _body(xf_ref, mask_ref, Ws_ref, bs_ref, Wq_ref, bq_ref, Wk_ref, bk_ref,
          Wv_ref, bv_ref, out_ref, frac_ref,
          s_buf, a_buf, en_ref):
    t = pl.program_id(1)

    @pl.when(t == 0)
    def _init():
        en_ref[0, 0] = 0.0

    xf = xf_ref[0]
    activ = jnp.dot(xf, Ws_ref[...],
                    preferred_element_type=jnp.float32) + bs_ref[...]
    activ_b = activ.astype(jnp.bfloat16)
    a_buf[pl.ds(t * TM, TM), :] = activ_b
    q = jnp.dot(activ_b, Wq_ref[...],
                preferred_element_type=jnp.float32) + bq_ref[...]
    k = jnp.dot(activ_b, Wk_ref[...],
                preferred_element_type=jnp.float32) + bk_ref[...]

    qk = (q * k).astype(jnp.bfloat16)
    lane = jax.lax.broadcasted_iota(jnp.int32, (HA, H), 0)
    head = jax.lax.broadcasted_iota(jnp.int32, (HA, H), 1)
    sel = (lane // A == head).astype(jnp.bfloat16)
    scores = jnp.dot(qk, sel,
                     preferred_element_type=jnp.float32) * (1.0 / (A ** 0.5))

    m = mask_ref[0]
    en_ref[0, 0] += jnp.sum(m)
    s_buf[pl.ds(t * TM, TM), :] = jnp.where(m > 0.5, scores, NEG)

    @pl.when(t == NT - 1)
    def _fin():
        s = s_buf[...]
        zmax = jnp.maximum(jnp.max(s, axis=0, keepdims=True), 0.0)
        ez = jnp.exp(s - zmax)
        den = jnp.sum(ez, axis=0, keepdims=True) + 1.0
        attn = (ez / den).astype(jnp.bfloat16)
        ta = jax.lax.dot_general(attn, a_buf[...], (((0,), (0,)), ((), ())),
                                 preferred_element_type=jnp.float32)
        full = jnp.dot(ta.astype(jnp.bfloat16), Wv_ref[...],
                       preferred_element_type=jnp.float32)
        sa = ((den - 1.0) / den).reshape(H, 1)
        full = full + sa * bv_ref[...]
        row = jax.lax.broadcasted_iota(jnp.int32, (H, HO), 0)
        col = jax.lax.broadcasted_iota(jnp.int32, (H, HO), 1)
        pick = (col // O == row).astype(jnp.float32)
        out_ref[0] = jnp.sum(full * pick, axis=0, keepdims=True)
        frac_ref[0] = jnp.broadcast_to(en_ref[0, 0] * (1.0 / M), (1, 1))


@jax.jit
def kernel(x, Ws, bs, Wq, bq, Wk, bk, Wv, bv):
    xf = x[:, : M * D].reshape(B, M, D).astype(jnp.bfloat16)
    mask = x[:, M * D:].reshape(B, M, 1)
    grid = (B, NT)
    out_main, frac = pl.pallas_call(
        _body,
        grid=grid,
        in_specs=[
            pl.BlockSpec((1, TM, D), lambda b, t: (b, t, 0)),
            pl.BlockSpec((1, TM, 1), lambda b, t: (b, t, 0)),
            pl.BlockSpec((D, D), lambda b, t: (0, 0)),
            pl.BlockSpec((1, D), lambda b, t: (0, 0)),
            pl.BlockSpec((D, HA), lambda b, t: (0, 0)),
            pl.BlockSpec((1, HA), lambda b, t: (0, 0)),
            pl.BlockSpec((D, HA), lambda b, t: (0, 0)),
            pl.BlockSpec((1, HA), lambda b, t: (0, 0)),
            pl.BlockSpec((D, HO), lambda b, t: (0, 0)),
            pl.BlockSpec((1, HO), lambda b, t: (0, 0)),
        ],
        out_specs=[
            pl.BlockSpec((1, 1, HO), lambda b, t: (b, 0, 0)),
            pl.BlockSpec((1, 1, 1), lambda b, t: (b, 0, 0)),
        ],
        out_shape=[
            jax.ShapeDtypeStruct((B, 1, HO), jnp.float32),
            jax.ShapeDtypeStruct((B, 1, 1), jnp.float32),
        ],
        scratch_shapes=[
            pltpu.VMEM((M, H), jnp.float32),
            pltpu.VMEM((M, D), jnp.bfloat16),
            pltpu.SMEM((1, 1), jnp.float32),
        ],
    )(xf, mask, Ws.astype(jnp.bfloat16), bs.reshape(1, D),
      Wq.astype(jnp.bfloat16), bq.reshape(1, HA),
      Wk.astype(jnp.bfloat16), bk.reshape(1, HA),
      Wv.astype(jnp.bfloat16), bv.reshape(1, HO))
    return jnp.concatenate([out_main.reshape(B, HO), frac.reshape(B, 1)],
                           axis=1)


# speedup vs baseline: 1.2938x; 1.0022x over previous
import jax
import jax.numpy as jnp
from jax.experimental import pallas as pl
from jax.experimental.pallas import tpu as pltpu

B = 16
M = 2048
D = 256
H = 8
A = 64
O = 64
HA = H * A
HO = H * O
NEG = -1e30


def _body(xf_ref, mask_ref, Ws_ref, bs_ref, Wq_ref, bq_ref, Wk_ref, bk_ref,
          Wv_ref, bv_ref, out_ref, frac_ref):
    xf = xf_ref[0]
    activ = jnp.dot(xf, Ws_ref[...],
                    preferred_element_type=jnp.float32) + bs_ref[...]
    activ_b = activ.astype(jnp.bfloat16)
    q = jnp.dot(activ_b, Wq_ref[...],
                preferred_element_type=jnp.float32) + bq_ref[...]
    k = jnp.dot(activ_b, Wk_ref[...],
                preferred_element_type=jnp.float32) + bk_ref[...]

    qk = (q * k).astype(jnp.bfloat16)
    lane = jax.lax.broadcasted_iota(jnp.int32, (HA, H), 0)
    head = jax.lax.broadcasted_iota(jnp.int32, (HA, H), 1)
    sel = (lane // A == head).astype(jnp.bfloat16)
    scores = jnp.dot(qk, sel,
                     preferred_element_type=jnp.float32) * (1.0 / (A ** 0.5))

    m = mask_ref[0]
    s = jnp.where(m > 0.5, scores, NEG)

    zmax = jnp.maximum(jnp.max(s, axis=0, keepdims=True), 0.0)
    ez = jnp.exp(s - zmax)
    den = jnp.sum(ez, axis=0, keepdims=True) + 1.0
    attn = (ez / den).astype(jnp.bfloat16)
    ta = jax.lax.dot_general(attn, activ_b, (((0,), (0,)), ((), ())),
                             preferred_element_type=jnp.float32)
    full = jnp.dot(ta.astype(jnp.bfloat16), Wv_ref[...],
                   preferred_element_type=jnp.float32)
    sa = ((den - 1.0) / den).reshape(H, 1)
    full = full + sa * bv_ref[...]
    row = jax.lax.broadcasted_iota(jnp.int32, (H, HO), 0)
    col = jax.lax.broadcasted_iota(jnp.int32, (H, HO), 1)
    pick = (col // O == row).astype(jnp.float32)
    out_ref[0] = jnp.sum(full * pick, axis=0, keepdims=True)
    frac_ref[0] = jnp.sum(m, axis=0, keepdims=True) * (1.0 / M)


@jax.jit
def kernel(x, Ws, bs, Wq, bq, Wk, bk, Wv, bv):
    xf = x[:, : M * D].reshape(B, M, D).astype(jnp.bfloat16)
    mask = x[:, M * D:].reshape(B, M, 1)
    out_main, frac = pl.pallas_call(
        _body,
        grid=(B,),
        in_specs=[
            pl.BlockSpec((1, M, D), lambda b: (b, 0, 0)),
            pl.BlockSpec((1, M, 1), lambda b: (b, 0, 0)),
            pl.BlockSpec((D, D), lambda b: (0, 0)),
            pl.BlockSpec((1, D), lambda b: (0, 0)),
            pl.BlockSpec((D, HA), lambda b: (0, 0)),
            pl.BlockSpec((1, HA), lambda b: (0, 0)),
            pl.BlockSpec((D, HA), lambda b: (0, 0)),
            pl.BlockSpec((1, HA), lambda b: (0, 0)),
            pl.BlockSpec((D, HO), lambda b: (0, 0)),
            pl.BlockSpec((1, HO), lambda b: (0, 0)),
        ],
        out_specs=[
            pl.BlockSpec((1, 1, HO), lambda b: (b, 0, 0)),
            pl.BlockSpec((1, 1, 1), lambda b: (b, 0, 0)),
        ],
        out_shape=[
            jax.ShapeDtypeStruct((B, 1, HO), jnp.float32),
            jax.ShapeDtypeStruct((B, 1, 1), jnp.float32),
        ],
    )(xf, mask, Ws.astype(jnp.bfloat16), bs.reshape(1, D),
      Wq.astype(jnp.bfloat16), bq.reshape(1, HA),
      Wk.astype(jnp.bfloat16), bk.reshape(1, HA),
      Wv.astype(jnp.bfloat16), bv.reshape(1, HO))
    return jnp.concatenate([out_main.reshape(B, HO), frac.reshape(B, 1)],
                           axis=1)
